# baseline (speedup 1.0000x reference)
import functools

import jax
import jax.numpy as jnp
from jax import lax
from jax.experimental import pallas as pl
from jax.experimental.pallas import tpu as pltpu
from jax.experimental.pallas import tpu_sc as plsc

N_NODES = 10000
NP = 10240
LANES = 16
NTILES = 16
NCORES = 2
F = 128
BW = 128
FN = 16
R = 512


def _loop(lo, hi, body):
    lax.fori_loop(lo, hi, lambda i, c: (body(i), c)[1], None)


def _make_deg(ep):
    epw = ep // (NCORES * NTILES)
    mesh = plsc.VectorSubcoreMesh(core_axis_name="c", subcore_axis_name="s")

    @functools.partial(
        pl.kernel,
        out_type=[jax.ShapeDtypeStruct((32, NP), jnp.float32),
                  jax.ShapeDtypeStruct((32, NP), jnp.float32)],
        mesh=mesh,
        scratch_types=[pltpu.VMEM((NP,), jnp.float32),
                       pltpu.VMEM((NP,), jnp.float32),
                       pltpu.VMEM((epw,), jnp.int32),
                       pltpu.VMEM((epw,), jnp.int32)],
        compiler_params=pltpu.CompilerParams(needs_layout_passes=False),
    )
    def deg_kernel(src_hbm, dst_hbm, dso_hbm, dsd_hbm, acc_s, acc_d, isrc, idst):
        c = lax.axis_index("c")
        s = lax.axis_index("s")
        w = c * NTILES + s
        zero = jnp.zeros((LANES,), jnp.float32)

        def z(i):
            acc_s[pl.ds(i * LANES, LANES)] = zero
            acc_d[pl.ds(i * LANES, LANES)] = zero
        _loop(0, NP // LANES, z)

        pltpu.sync_copy(src_hbm.at[pl.ds(w * epw, epw)], isrc)
        pltpu.sync_copy(dst_hbm.at[pl.ds(w * epw, epw)], idst)
        ones = jnp.ones((LANES,), jnp.float32)

        def body(i):
            sv = isrc[pl.ds(i * LANES, LANES)]
            dv = idst[pl.ds(i * LANES, LANES)]
            plsc.addupdate_scatter(acc_s, [sv], ones)
            plsc.addupdate_scatter(acc_d, [dv], ones)
        _loop(0, epw // LANES, body)

        pltpu.sync_copy(acc_s, dso_hbm.at[w])
        pltpu.sync_copy(acc_d, dsd_hbm.at[w])

    return deg_kernel


def _make_swide(ep, ncpc):
    ept = ep // NTILES
    nb = ept // BW
    C = NCORES * ncpc
    rpt = NP // NTILES
    mesh = plsc.VectorSubcoreMesh(core_axis_name="c", subcore_axis_name="s")

    @functools.partial(
        pl.kernel,
        out_type=jax.ShapeDtypeStruct((C * NP, F), jnp.float32),
        mesh=mesh,
        scratch_types=[pltpu.VMEM((nb, BW), jnp.int32),
                       [pltpu.VMEM((BW,), jnp.int32) for _ in range(4)],
                       pltpu.VMEM((BW, F), jnp.float32),
                       pltpu.VMEM((BW, F), jnp.float32),
                       pltpu.VMEM_SHARED((NP, F), jnp.float32),
                       [pltpu.SemaphoreType.DMA for _ in range(4)],
                       pltpu.SemaphoreType.DMA,
                       pltpu.SemaphoreType.DMA,
                       pltpu.SemaphoreType.DMA,
                       pltpu.SemaphoreType.DMA],
    )
    def swide(src4_hbm, dst3_hbm, h_hbm, out_hbm, idst, ix, rows_a,
              rows_b, acc, isem, g_a, g_b, s_a, s_b):
        c = lax.axis_index("c")
        s = lax.axis_index("s")
        pltpu.sync_copy(dst3_hbm.at[s], idst)
        zero = jnp.zeros((LANES,), jnp.float32)
        base_row = s * rpt

        for j in range(ncpc):
            cid = c * ncpc + j

            def zr(i):
                rows_a[i // (F // LANES),
                       pl.ds((i % (F // LANES)) * LANES, LANES)] = zero
            _loop(0, BW * (F // LANES), zr)
            for k in range(rpt // BW):
                pltpu.sync_copy(rows_a, acc.at[pl.ds(base_row + k * BW, BW)])
            plsc.subcore_barrier()

            def prefetch(q, b):
                pltpu.async_copy(src4_hbm.at[cid, s, b], ix[q], isem[q])

            def wait_idx(q, b):
                pltpu.make_async_copy(src4_hbm.at[cid, s, b], ix[q],
                                      isem[q]).wait()

            def gather(q, rows, gsem, b):
                pltpu.async_copy(h_hbm.at[ix[q]], rows, gsem)

            def wait_gather(q, rows, gsem):
                pltpu.make_async_copy(h_hbm.at[ix[q]], rows, gsem).wait()

            def scatter(rows, ssem, b):
                pltpu.async_copy(rows, acc.at[idst.at[b]], ssem, add=True)

            def wait_scatter(rows, ssem, b):
                pltpu.make_async_copy(rows, acc.at[idst.at[b]], ssem).wait()

            for q in range(4):
                prefetch(q, q)
            wait_idx(0, 0)
            gather(0, rows_a, g_a, 0)
            wait_idx(1, 1)
            gather(1, rows_b, g_b, 1)

            def quad(t):
                b = 4 * t
                wait_gather(0, rows_a, g_a)
                scatter(rows_a, s_a, b)
                wait_gather(1, rows_b, g_b)
                scatter(rows_b, s_b, b + 1)
                wait_idx(2, b + 2)
                wait_scatter(rows_a, s_a, b)
                gather(2, rows_a, g_a, b + 2)

                @pl.when(b + 4 < nb)
                def _():
                    prefetch(0, b + 4)
                wait_idx(3, b + 3)
                wait_scatter(rows_b, s_b, b + 1)
                gather(3, rows_b, g_b, b + 3)

                @pl.when(b + 5 < nb)
                def _():
                    prefetch(1, b + 5)
                wait_gather(2, rows_a, g_a)
                scatter(rows_a, s_a, b + 2)
                wait_gather(3, rows_b, g_b)
                scatter(rows_b, s_b, b + 3)
                wait_scatter(rows_a, s_a, b + 2)

                @pl.when(b + 4 < nb)
                def _():
                    wait_idx(0, b + 4)
                    gather(0, rows_a, g_a, b + 4)

                @pl.when(b + 6 < nb)
                def _():
                    prefetch(2, b + 6)
                wait_scatter(rows_b, s_b, b + 3)

                @pl.when(b + 5 < nb)
                def _():
                    wait_idx(1, b + 5)
                    gather(1, rows_b, g_b, b + 5)

                @pl.when(b + 7 < nb)
                def _():
                    prefetch(3, b + 7)
            _loop(0, nb // 4, quad)
            plsc.subcore_barrier()

            base = cid * NP
            for k in range(rpt // F):
                pltpu.sync_copy(acc.at[pl.ds(base_row + k * F, F)],
                                out_hbm.at[pl.ds(base + base_row + k * F, F)])
            if j + 1 < ncpc:
                plsc.subcore_barrier()

    return swide


def _make_snarrow(ep):
    epw = ep // (NCORES * NTILES)
    mesh = plsc.VectorSubcoreMesh(core_axis_name="c", subcore_axis_name="s")

    @functools.partial(
        pl.kernel,
        out_type=jax.ShapeDtypeStruct((32, NP * 2), jnp.float32),
        mesh=mesh,
        scratch_types=[pltpu.VMEM((NP * 2,), jnp.float32),
                       pltpu.VMEM((NP * 2,), jnp.float32),
                       pltpu.VMEM((epw,), jnp.int32),
                       pltpu.VMEM((epw,), jnp.int32)],
        compiler_params=pltpu.CompilerParams(needs_layout_passes=False),
    )
    def snarrow(g_hbm, src_hbm, dst_hbm, out_hbm, gbuf, acc, isrc, idst):
        c = lax.axis_index("c")
        s = lax.axis_index("s")
        w = c * NTILES + s
        zero = jnp.zeros((LANES,), jnp.float32)

        def z(i):
            acc[pl.ds(i * LANES, LANES)] = zero
        _loop(0, NP * 2 // LANES, z)

        pltpu.sync_copy(g_hbm, gbuf)
        pltpu.sync_copy(src_hbm.at[pl.ds(w * epw, epw)], isrc)
        pltpu.sync_copy(dst_hbm.at[pl.ds(w * epw, epw)], idst)

        def body(i):
            sv = isrc[pl.ds(i * LANES, LANES)]
            dv = idst[pl.ds(i * LANES, LANES)]
            sv2 = sv + sv
            dv2 = dv + dv
            v0 = plsc.load_gather(gbuf, [sv2])
            plsc.addupdate_scatter(acc, [dv2], v0)
            v1 = plsc.load_gather(gbuf, [sv2 + 1])
            plsc.addupdate_scatter(acc, [dv2 + 1], v1)
        _loop(0, epw // LANES, body)

        pltpu.sync_copy(acc, out_hbm.at[w])

    return snarrow


def _stage_a_body(x_ref, dso_ref, dsd_ref, xs_ref, dns_ref, dnd_ref):
    i = pl.program_id(0)
    row = i * R + lax.broadcasted_iota(jnp.int32, (R, 1), 0)
    valid = (row < N_NODES).astype(jnp.float32)
    dn_s = lax.rsqrt(jnp.maximum(jnp.sum(dso_ref[...], axis=0), 1.0))[:, None] * valid
    dn_d = lax.rsqrt(jnp.maximum(jnp.sum(dsd_ref[...], axis=0), 1.0))[:, None] * valid
    xs = x_ref[...] * dn_s
    xs_ref[0] = xs[:, :F]
    xs_ref[1] = xs[:, F:]
    dns_ref[...] = dn_s
    dnd_ref[...] = dn_d


_stage_a = pl.pallas_call(
    _stage_a_body,
    grid=(NP // R,),
    in_specs=[pl.BlockSpec((R, 256), lambda i: (i, 0)),
              pl.BlockSpec((32, R), lambda i: (0, i)),
              pl.BlockSpec((32, R), lambda i: (0, i))],
    out_specs=[pl.BlockSpec((2, R, F), lambda i: (0, i, 0)),
               pl.BlockSpec((R, 1), lambda i: (i, 0)),
               pl.BlockSpec((R, 1), lambda i: (i, 0))],
    out_shape=[jax.ShapeDtypeStruct((2, NP, F), jnp.float32),
               jax.ShapeDtypeStruct((NP, 1), jnp.float32),
               jax.ShapeDtypeStruct((NP, 1), jnp.float32)],
)


def _stage_b_body(a0_ref, dnd_ref, dns_ref, w_ref, b_ref, out_ref):
    a = jnp.concatenate([a0_ref[0], a0_ref[1]], axis=1) * dnd_ref[...]
    h = jnp.dot(a, w_ref[...], preferred_element_type=jnp.float32) + b_ref[...]
    h = jnp.maximum(h, 0.0) * dns_ref[...]
    for k in range(4):
        out_ref[k] = h[:, k * F:(k + 1) * F]


_stage_b = pl.pallas_call(
    _stage_b_body,
    grid=(NP // R,),
    in_specs=[pl.BlockSpec((2, R, F), lambda i: (0, i, 0)),
              pl.BlockSpec((R, 1), lambda i: (i, 0)),
              pl.BlockSpec((R, 1), lambda i: (i, 0)),
              pl.BlockSpec((256, 512), lambda i: (0, 0)),
              pl.BlockSpec((1, 512), lambda i: (0, 0))],
    out_specs=pl.BlockSpec((4, R, F), lambda i: (0, i, 0)),
    out_shape=jax.ShapeDtypeStruct((4, NP, F), jnp.float32),
)


def _stage_c_body(a1_ref, dnd_ref, dns_ref, w1_ref, b1_ref, wf_ref, out_ref):
    a = jnp.concatenate([a1_ref[k] for k in range(4)], axis=1) * dnd_ref[...]
    h = jnp.maximum(
        jnp.dot(a, w1_ref[...], preferred_element_type=jnp.float32) + b1_ref[...],
        0.0)
    out_ref[...] = jnp.dot(h, wf_ref[...],
                           preferred_element_type=jnp.float32) * dns_ref[...]


_stage_c = pl.pallas_call(
    _stage_c_body,
    grid=(NP // R,),
    in_specs=[pl.BlockSpec((4, R, F), lambda i: (0, i, 0)),
              pl.BlockSpec((R, 1), lambda i: (i, 0)),
              pl.BlockSpec((R, 1), lambda i: (i, 0)),
              pl.BlockSpec((512, 512), lambda i: (0, 0)),
              pl.BlockSpec((1, 512), lambda i: (0, 0)),
              pl.BlockSpec((512, 2), lambda i: (0, 0))],
    out_specs=pl.BlockSpec((R, 2), lambda i: (i, 0)),
    out_shape=jax.ShapeDtypeStruct((NP, 2), jnp.float32),
)


def _stage_d_body(a2_ref, dnd_ref, bf_ref, out_ref):
    acc = jnp.sum(a2_ref[...], axis=0)
    out_ref[...] = acc * dnd_ref[...] + bf_ref[...]


_stage_d = pl.pallas_call(
    _stage_d_body,
    grid=(NP // R,),
    in_specs=[pl.BlockSpec((32, R, 2), lambda i: (0, i, 0)),
              pl.BlockSpec((R, 1), lambda i: (i, 0)),
              pl.BlockSpec((1, 2), lambda i: (0, 0))],
    out_specs=pl.BlockSpec((R, 2), lambda i: (i, 0)),
    out_shape=jax.ShapeDtypeStruct((NP, 2), jnp.float32),
)


def kernel(x, edge_index, W0, b0, W1, b1, Wf, bf):
    src = edge_index[0].astype(jnp.int32)
    dst = edge_index[1].astype(jnp.int32)
    e = src.shape[0]
    ep = -(-e // 4096) * 4096
    pad = ep - e
    src_p = jnp.concatenate([src, jnp.full((pad,), N_NODES, jnp.int32)])
    dst_p = jnp.concatenate([dst, jnp.full((pad,), N_NODES, jnp.int32)])
    nb = ep // NTILES // BW
    dst3w = dst_p.reshape(NTILES, nb, BW)
    off2 = (jnp.arange(2, dtype=jnp.int32) * NP)[:, None]
    off4 = (jnp.arange(4, dtype=jnp.int32) * NP)[:, None]
    src4_2 = (src_p[None, :] + off2).reshape(2, NTILES, nb, BW)
    src4_4 = (src_p[None, :] + off4).reshape(4, NTILES, nb, BW)

    x_p = jnp.pad(x, ((0, NP - N_NODES), (0, 0)))

    dso, dsd = _make_deg(ep)(src_p, dst_p)
    xs, dns, dnd = _stage_a(x_p, dso, dsd)
    a0 = _make_swide(ep, 1)(src4_2, dst3w, xs.reshape(2 * NP, F))
    h1s = _stage_b(a0.reshape(2, NP, F), dnd, dns, W0, b0.reshape(1, 512))
    a1 = _make_swide(ep, 2)(src4_4, dst3w, h1s.reshape(4 * NP, F))
    gs = _stage_c(a1.reshape(4, NP, F), dnd, dns, W1, b1.reshape(1, 512), Wf)
    a2 = _make_snarrow(ep)(gs.reshape(NP * 2), src_p, dst_p)
    out = _stage_d(a2.reshape(32, NP, 2), dnd, bf.reshape(1, 2))
    return out[:N_NODES]


# speedup vs baseline: 1.0647x; 1.0647x over previous
import functools

import jax
import jax.numpy as jnp
from jax import lax
from jax.experimental import pallas as pl
from jax.experimental.pallas import tpu as pltpu
from jax.experimental.pallas import tpu_sc as plsc

N_NODES = 10000
NP = 10240
LANES = 16
NTILES = 16
NCORES = 2
F = 128
BW = 128
FN = 16
R = 512


def _loop(lo, hi, body):
    lax.fori_loop(lo, hi, lambda i, c: (body(i), c)[1], None)


def _make_deg(ep):
    epw = ep // (NCORES * NTILES)
    mesh = plsc.VectorSubcoreMesh(core_axis_name="c", subcore_axis_name="s")

    @functools.partial(
        pl.kernel,
        out_type=[jax.ShapeDtypeStruct((32, NP), jnp.float32),
                  jax.ShapeDtypeStruct((32, NP), jnp.float32)],
        mesh=mesh,
        scratch_types=[pltpu.VMEM((NP,), jnp.float32),
                       pltpu.VMEM((NP,), jnp.float32),
                       pltpu.VMEM((epw,), jnp.int32),
                       pltpu.VMEM((epw,), jnp.int32)],
        compiler_params=pltpu.CompilerParams(needs_layout_passes=False),
    )
    def deg_kernel(src_hbm, dst_hbm, dso_hbm, dsd_hbm, acc_s, acc_d, isrc, idst):
        c = lax.axis_index("c")
        s = lax.axis_index("s")
        w = c * NTILES + s
        zero = jnp.zeros((LANES,), jnp.float32)

        def z(i):
            acc_s[pl.ds(i * LANES, LANES)] = zero
            acc_d[pl.ds(i * LANES, LANES)] = zero
        _loop(0, NP // LANES, z)

        pltpu.sync_copy(src_hbm.at[pl.ds(w * epw, epw)], isrc)
        pltpu.sync_copy(dst_hbm.at[pl.ds(w * epw, epw)], idst)
        ones = jnp.ones((LANES,), jnp.float32)

        def body(i):
            sv = isrc[pl.ds(i * LANES, LANES)]
            dv = idst[pl.ds(i * LANES, LANES)]
            plsc.addupdate_scatter(acc_s, [sv], ones)
            plsc.addupdate_scatter(acc_d, [dv], ones)
        _loop(0, epw // LANES, body)

        pltpu.sync_copy(acc_s, dso_hbm.at[w])
        pltpu.sync_copy(acc_d, dsd_hbm.at[w])

    return deg_kernel


def _make_swide(ep, ncpc):
    ept = ep // NTILES
    nb = ept // BW
    C = NCORES * ncpc
    rpt = NP // NTILES
    mesh = plsc.VectorSubcoreMesh(core_axis_name="c", subcore_axis_name="s")

    @functools.partial(
        pl.kernel,
        out_type=jax.ShapeDtypeStruct((C * NP, F), jnp.float32),
        mesh=mesh,
        scratch_types=[pltpu.VMEM((nb, BW), jnp.int32),
                       [pltpu.VMEM((BW,), jnp.int32) for _ in range(4)],
                       pltpu.VMEM((BW, F), jnp.float32),
                       pltpu.VMEM((BW, F), jnp.float32),
                       pltpu.VMEM_SHARED((NP, F), jnp.float32),
                       [pltpu.SemaphoreType.DMA for _ in range(4)],
                       pltpu.SemaphoreType.DMA,
                       pltpu.SemaphoreType.DMA,
                       pltpu.SemaphoreType.DMA,
                       pltpu.SemaphoreType.DMA],
    )
    def swide(src4_hbm, dst3_hbm, h_hbm, out_hbm, idst, ix, rows_a,
              rows_b, acc, isem, g_a, g_b, s_a, s_b):
        c = lax.axis_index("c")
        s = lax.axis_index("s")
        pltpu.sync_copy(dst3_hbm.at[s], idst)
        zero = jnp.zeros((LANES,), jnp.float32)
        base_row = s * rpt

        for j in range(ncpc):
            cid = c * ncpc + j

            def zr(i):
                rows_a[i // (F // LANES),
                       pl.ds((i % (F // LANES)) * LANES, LANES)] = zero
            _loop(0, BW * (F // LANES), zr)
            for k in range(rpt // BW):
                pltpu.sync_copy(rows_a, acc.at[pl.ds(base_row + k * BW, BW)])
            plsc.subcore_barrier()

            def stage(q, b):
                pltpu.async_copy(src4_hbm.at[cid, s, b], ix[q], isem[q])

            def gather(q, rows, gsem, b):
                pltpu.make_async_copy(src4_hbm.at[cid, s, b], ix[q],
                                      isem[q]).wait()
                pltpu.async_copy(h_hbm.at[ix[q]], rows, gsem)

            stage(0, 0)
            gather(0, rows_a, g_a, 0)
            stage(1, 1)

            def pair(p):
                b0 = 2 * p
                gather(1, rows_b, g_b, b0 + 1)

                @pl.when(b0 + 2 < nb)
                def _():
                    stage(0, b0 + 2)
                pltpu.make_async_copy(h_hbm.at[ix[0]], rows_a, g_a).wait()
                pltpu.sync_copy(rows_a, acc.at[idst.at[b0]], add=True)

                @pl.when(b0 + 2 < nb)
                def _():
                    gather(0, rows_a, g_a, b0 + 2)

                @pl.when(b0 + 3 < nb)
                def _():
                    stage(1, b0 + 3)
                pltpu.make_async_copy(h_hbm.at[ix[1]], rows_b, g_b).wait()
                pltpu.sync_copy(rows_b, acc.at[idst.at[b0 + 1]], add=True)
            _loop(0, nb // 2, pair)
            plsc.subcore_barrier()

            base = cid * NP
            for k in range(rpt // F):
                pltpu.sync_copy(acc.at[pl.ds(base_row + k * F, F)],
                                out_hbm.at[pl.ds(base + base_row + k * F, F)])
            if j + 1 < ncpc:
                plsc.subcore_barrier()

    return swide


def _make_snarrow(ep):
    epw = ep // (NCORES * NTILES)
    mesh = plsc.VectorSubcoreMesh(core_axis_name="c", subcore_axis_name="s")

    @functools.partial(
        pl.kernel,
        out_type=jax.ShapeDtypeStruct((32, NP * 2), jnp.float32),
        mesh=mesh,
        scratch_types=[pltpu.VMEM((NP * 2,), jnp.float32),
                       pltpu.VMEM((NP * 2,), jnp.float32),
                       pltpu.VMEM((epw,), jnp.int32),
                       pltpu.VMEM((epw,), jnp.int32)],
        compiler_params=pltpu.CompilerParams(needs_layout_passes=False),
    )
    def snarrow(g_hbm, src_hbm, dst_hbm, out_hbm, gbuf, acc, isrc, idst):
        c = lax.axis_index("c")
        s = lax.axis_index("s")
        w = c * NTILES + s
        zero = jnp.zeros((LANES,), jnp.float32)

        def z(i):
            acc[pl.ds(i * LANES, LANES)] = zero
        _loop(0, NP * 2 // LANES, z)

        pltpu.sync_copy(g_hbm, gbuf)
        pltpu.sync_copy(src_hbm.at[pl.ds(w * epw, epw)], isrc)
        pltpu.sync_copy(dst_hbm.at[pl.ds(w * epw, epw)], idst)

        def body(i):
            sv = isrc[pl.ds(i * LANES, LANES)]
            dv = idst[pl.ds(i * LANES, LANES)]
            sv2 = sv + sv
            dv2 = dv + dv
            v0 = plsc.load_gather(gbuf, [sv2])
            plsc.addupdate_scatter(acc, [dv2], v0)
            v1 = plsc.load_gather(gbuf, [sv2 + 1])
            plsc.addupdate_scatter(acc, [dv2 + 1], v1)
        _loop(0, epw // LANES, body)

        pltpu.sync_copy(acc, out_hbm.at[w])

    return snarrow


def _stage_a_body(x_ref, dso_ref, dsd_ref, xs_ref, dns_ref, dnd_ref):
    i = pl.program_id(0)
    row = i * R + lax.broadcasted_iota(jnp.int32, (R, 1), 0)
    valid = (row < N_NODES).astype(jnp.float32)
    dn_s = lax.rsqrt(jnp.maximum(jnp.sum(dso_ref[...], axis=0), 1.0))[:, None] * valid
    dn_d = lax.rsqrt(jnp.maximum(jnp.sum(dsd_ref[...], axis=0), 1.0))[:, None] * valid
    xs = x_ref[...] * dn_s
    xs_ref[0] = xs[:, :F]
    xs_ref[1] = xs[:, F:]
    dns_ref[...] = dn_s
    dnd_ref[...] = dn_d


_stage_a = pl.pallas_call(
    _stage_a_body,
    grid=(NP // R,),
    in_specs=[pl.BlockSpec((R, 256), lambda i: (i, 0)),
              pl.BlockSpec((32, R), lambda i: (0, i)),
              pl.BlockSpec((32, R), lambda i: (0, i))],
    out_specs=[pl.BlockSpec((2, R, F), lambda i: (0, i, 0)),
               pl.BlockSpec((R, 1), lambda i: (i, 0)),
               pl.BlockSpec((R, 1), lambda i: (i, 0))],
    out_shape=[jax.ShapeDtypeStruct((2, NP, F), jnp.float32),
               jax.ShapeDtypeStruct((NP, 1), jnp.float32),
               jax.ShapeDtypeStruct((NP, 1), jnp.float32)],
)


def _stage_b_body(a0_ref, dnd_ref, dns_ref, w_ref, b_ref, out_ref):
    a = jnp.concatenate([a0_ref[0], a0_ref[1]], axis=1) * dnd_ref[...]
    h = jnp.dot(a, w_ref[...], preferred_element_type=jnp.float32) + b_ref[...]
    h = jnp.maximum(h, 0.0) * dns_ref[...]
    for k in range(4):
        out_ref[k] = h[:, k * F:(k + 1) * F]


_stage_b = pl.pallas_call(
    _stage_b_body,
    grid=(NP // R,),
    in_specs=[pl.BlockSpec((2, R, F), lambda i: (0, i, 0)),
              pl.BlockSpec((R, 1), lambda i: (i, 0)),
              pl.BlockSpec((R, 1), lambda i: (i, 0)),
              pl.BlockSpec((256, 512), lambda i: (0, 0)),
              pl.BlockSpec((1, 512), lambda i: (0, 0))],
    out_specs=pl.BlockSpec((4, R, F), lambda i: (0, i, 0)),
    out_shape=jax.ShapeDtypeStruct((4, NP, F), jnp.float32),
)


def _stage_c_body(a1_ref, dnd_ref, dns_ref, w1_ref, b1_ref, wf_ref, out_ref):
    a = jnp.concatenate([a1_ref[k] for k in range(4)], axis=1) * dnd_ref[...]
    h = jnp.maximum(
        jnp.dot(a, w1_ref[...], preferred_element_type=jnp.float32) + b1_ref[...],
        0.0)
    out_ref[...] = jnp.dot(h, wf_ref[...],
                           preferred_element_type=jnp.float32) * dns_ref[...]


_stage_c = pl.pallas_call(
    _stage_c_body,
    grid=(NP // R,),
    in_specs=[pl.BlockSpec((4, R, F), lambda i: (0, i, 0)),
              pl.BlockSpec((R, 1), lambda i: (i, 0)),
              pl.BlockSpec((R, 1), lambda i: (i, 0)),
              pl.BlockSpec((512, 512), lambda i: (0, 0)),
              pl.BlockSpec((1, 512), lambda i: (0, 0)),
              pl.BlockSpec((512, 2), lambda i: (0, 0))],
    out_specs=pl.BlockSpec((R, 2), lambda i: (i, 0)),
    out_shape=jax.ShapeDtypeStruct((NP, 2), jnp.float32),
)


def _stage_d_body(a2_ref, dnd_ref, bf_ref, out_ref):
    acc = jnp.sum(a2_ref[...], axis=0)
    out_ref[...] = acc * dnd_ref[...] + bf_ref[...]


_stage_d = pl.pallas_call(
    _stage_d_body,
    grid=(NP // R,),
    in_specs=[pl.BlockSpec((32, R, 2), lambda i: (0, i, 0)),
              pl.BlockSpec((R, 1), lambda i: (i, 0)),
              pl.BlockSpec((1, 2), lambda i: (0, 0))],
    out_specs=pl.BlockSpec((R, 2), lambda i: (i, 0)),
    out_shape=jax.ShapeDtypeStruct((NP, 2), jnp.float32),
)


def kernel(x, edge_index, W0, b0, W1, b1, Wf, bf):
    src = edge_index[0].astype(jnp.int32)
    dst = edge_index[1].astype(jnp.int32)
    e = src.shape[0]
    ep = -(-e // 4096) * 4096
    pad = ep - e
    src_p = jnp.concatenate([src, jnp.full((pad,), N_NODES, jnp.int32)])
    dst_p = jnp.concatenate([dst, jnp.full((pad,), N_NODES, jnp.int32)])
    nb = ep // NTILES // BW
    dst3w = dst_p.reshape(NTILES, nb, BW)
    off2 = (jnp.arange(2, dtype=jnp.int32) * NP)[:, None]
    off4 = (jnp.arange(4, dtype=jnp.int32) * NP)[:, None]
    src4_2 = (src_p[None, :] + off2).reshape(2, NTILES, nb, BW)
    src4_4 = (src_p[None, :] + off4).reshape(4, NTILES, nb, BW)

    x_p = jnp.pad(x, ((0, NP - N_NODES), (0, 0)))

    dso, dsd = _make_deg(ep)(src_p, dst_p)
    xs, dns, dnd = _stage_a(x_p, dso, dsd)
    a0 = _make_swide(ep, 1)(src4_2, dst3w, xs.reshape(2 * NP, F))
    h1s = _stage_b(a0.reshape(2, NP, F), dnd, dns, W0, b0.reshape(1, 512))
    a1 = _make_swide(ep, 2)(src4_4, dst3w, h1s.reshape(4 * NP, F))
    gs = _stage_c(a1.reshape(4, NP, F), dnd, dns, W1, b1.reshape(1, 512), Wf)
    a2 = _make_snarrow(ep)(gs.reshape(NP * 2), src_p, dst_p)
    out = _stage_d(a2.reshape(32, NP, 2), dnd, bf.reshape(1, 2))
    return out[:N_NODES]
